# baseline (speedup 1.0000x reference)
import jax, jax.numpy as jnp
import numpy as np

def kernel(x, edge_index, edge_vec, w1, b1, w2, b2, w3, b3, lw0, lw1, lw2, lb):
    src = edge_index[0]; dst = edge_index[1]
    d = jnp.linalg.norm(edge_vec, axis=-1)
    r = d[:, None]
    n = edge_vec / (r + 1e-12)
    xx, yy, zz = n[..., 0], n[..., 1], n[..., 2]
    c0 = 0.28209479177387814; c1 = 0.4886025119029199; c2 = 1.0925484305920792
    c20 = 0.31539156525252005; c22 = 0.5462742152960396
    sh = jnp.stack([jnp.full_like(xx, c0), c1*yy, c1*zz, c1*xx,
                    c2*xx*yy, c2*yy*zz, c20*(3.0*zz*zz-1.0), c2*xx*zz,
                    c22*(xx*xx-yy*yy)], axis=-1)
    centers = jnp.linspace(0.0, 5.0, 16).astype(x.dtype)
    widths = jnp.full((16,), 5.0/16, dtype=x.dtype)
    rbf = jnp.exp(-(d[:, None]-centers[None, :])**2/(2.0*widths[None, :]**2))
    cut = 0.5*(1.0+jnp.cos(jnp.pi*d/5.0))*(d < 5.0).astype(x.dtype)
    h = jax.nn.silu(rbf @ w1.T + b1)
    h = jax.nn.silu(h @ w2.T + b2)
    rw = (h @ w3.T + b3).reshape(-1, 32, 32)
    rw = rw * cut[:, None, None]
    x_src = jnp.take(x, src, axis=0)
    msg = x_src * sh[:, None, :]
    msg = jnp.einsum('eoi,eih->eoh', rw, msg)
    out = jnp.zeros((x.shape[0], 32, 9), dtype=x.dtype).at[dst].add(msg)
    res = jnp.zeros_like(out)
    idx = 0
    for l, w in zip(range(3), (lw0, lw1, lw2)):
        n_m = 2*l+1
        res = res.at[..., idx:idx+n_m].set(jnp.einsum('nim,oi->nom', out[..., idx:idx+n_m], w))
        idx += n_m
    res = res.at[..., 0].add(lb[None, :])
    return res


# speedup vs baseline: 5.3825x; 5.3825x over previous
import functools

import jax
import jax.numpy as jnp
import numpy as np
from jax import lax
from jax.experimental import pallas as pl
from jax.experimental.pallas import tpu as pltpu
from jax.experimental.pallas import tpu_sc as plsc

N_NODES = 10000
IN_F = 32
OUT_F = 32
N_HARM = 9
F = IN_F * N_HARM
FP = 384
N_RBF = 16
CUTOFF = 5.0

NC = 2
NS = 16
NW = NC * NS

E_PAD = 102400
B_PER_W = E_PAD // NW
GCH = 128

EB = 512
NB = 1024

WN = 512
NWIN = 10240 // WN
CH2 = 1024
NCHUNK = E_PAD // CH2

_SH_C0 = 0.28209479177387814
_SH_C1 = 0.4886025119029199
_SH_C2 = 1.0925484305920792
_SH_C20 = 0.31539156525252005
_SH_C22 = 0.5462742152960396

_TM = np.zeros((N_HARM, FP), np.float32)
for _i in range(IN_F):
    for _h in range(N_HARM):
        _TM[_h, _i * N_HARM + _h] = 1.0
_RM = np.zeros((OUT_F, FP), np.float32)
for _o in range(OUT_F):
    for _h in range(N_HARM):
        _RM[_o, _o * N_HARM + _h] = 1.0


def _gather_body(x_hbm, src_hbm, out_hbm, idx_v, rows_v, sem):
    wid = lax.axis_index("s") * NC + lax.axis_index("c")
    base = wid * B_PER_W
    pltpu.sync_copy(src_hbm.at[wid], idx_v)

    def body(cc, carry):
        pltpu.async_copy(
            x_hbm.at[idx_v.at[pl.ds(cc * GCH, GCH)]], rows_v, sem
        ).wait()
        pltpu.sync_copy(rows_v, out_hbm.at[pl.ds(base + cc * GCH, GCH)])
        return carry

    lax.fori_loop(0, B_PER_W // GCH, body, 0)


_gather_call = functools.partial(
    pl.kernel,
    out_type=jax.ShapeDtypeStruct((E_PAD, FP), jnp.float32),
    mesh=plsc.VectorSubcoreMesh(
        core_axis_name="c", subcore_axis_name="s", num_cores=NC, num_subcores=NS
    ),
    scratch_types=[
        pltpu.VMEM((B_PER_W,), jnp.int32),
        pltpu.VMEM((GCH, FP), jnp.float32),
        pltpu.SemaphoreType.DMA,
    ],
)(_gather_body)


def _dense_body(ev_ref, xsrc_ref, w1t_ref, b1_ref, w2t_ref, b2_ref, w3pt_ref,
                b3p_ref, tm_ref, rm_ref, out_ref):
    ev = ev_ref[...]
    d2 = jnp.sum(ev * ev, axis=1, keepdims=True)
    dd = jnp.sqrt(d2)
    nv = ev / (dd + 1e-12)
    xn, yn, zn = nv[:, 0:1], nv[:, 1:2], nv[:, 2:3]
    sh = jnp.concatenate(
        [
            jnp.full_like(xn, _SH_C0),
            _SH_C1 * yn, _SH_C1 * zn, _SH_C1 * xn,
            _SH_C2 * xn * yn, _SH_C2 * yn * zn,
            _SH_C20 * (3.0 * zn * zn - 1.0),
            _SH_C2 * xn * zn, _SH_C22 * (xn * xn - yn * yn),
        ],
        axis=1,
    )
    centers = lax.broadcasted_iota(jnp.int32, (1, N_RBF), 1).astype(
        jnp.float32) * (CUTOFF / (N_RBF - 1))
    width = CUTOFF / N_RBF
    rbf = jnp.exp(-((dd - centers) ** 2) * (1.0 / (2.0 * width * width)))
    cut = 0.5 * (1.0 + jnp.cos(dd * (np.pi / CUTOFF)))
    cut = jnp.where(dd < CUTOFF, cut, 0.0)

    h1 = rbf @ w1t_ref[...] + b1_ref[...]
    h1 = h1 * (1.0 / (1.0 + jnp.exp(-h1)))
    h2 = h1 @ w2t_ref[...] + b2_ref[...]
    h2 = h2 * (1.0 / (1.0 + jnp.exp(-h2)))
    rw = (h2 @ w3pt_ref[...] + b3p_ref[...]) * cut

    tm = tm_ref[...]
    rm = rm_ref[...]
    xs = xsrc_ref[...] * (sh @ tm)

    acc = jnp.zeros((EB, FP), jnp.float32)
    for i in range(IN_F):
        a = rw[:, i * OUT_F:(i + 1) * OUT_F] @ rm
        b = xs[:, i * N_HARM:(i + 1) * N_HARM] @ tm
        acc = acc + a * b
    out_ref[...] = acc


def _dense_call(ev_p, x_src, w1t, b1r, w2t, b2r, w3pt, b3pr, tm, rm):
    grid = (E_PAD // EB,)
    full = lambda shape: pl.BlockSpec(shape, lambda b: (0, 0))
    return pl.pallas_call(
        _dense_body,
        grid=grid,
        in_specs=[
            pl.BlockSpec((EB, 3), lambda b: (b, 0)),
            pl.BlockSpec((EB, FP), lambda b: (b, 0)),
            full((N_RBF, 64)),
            full((1, 64)),
            full((64, 64)),
            full((1, 64)),
            full((64, 1024)),
            full((1, 1024)),
            full((N_HARM, FP)),
            full((OUT_F, FP)),
        ],
        out_specs=pl.BlockSpec((EB, FP), lambda b: (b, 0)),
        out_shape=jax.ShapeDtypeStruct((E_PAD, FP), jnp.float32),
    )(ev_p, x_src, w1t, b1r, w2t, b2r, w3pt, b3pr, tm, rm)


def _window_body(estart_ref, dstc_hbm, msg_hbm, out_ref, dstv, msgv, sem_d,
                 sem_m):
    w = pl.program_id(0)
    e0 = estart_ref[w]
    e1 = estart_ref[w + 1]
    c0 = e0 // CH2
    c1 = lax.div(e1 + CH2 - 1, CH2)
    base = w * WN
    out_ref[...] = jnp.zeros((WN, FP), jnp.float32)

    def chunk(c, carry):
        cp_d = pltpu.make_async_copy(dstc_hbm.at[pl.ds(c, 1)], dstv, sem_d)
        cp_m = pltpu.make_async_copy(
            msg_hbm.at[pl.ds(c * CH2, CH2)], msgv, sem_m)
        cp_d.start()
        cp_m.start()
        cp_d.wait()
        cp_m.wait()
        node_ids = lax.broadcasted_iota(jnp.int32, (WN, CH2), 0) + base
        oh = jnp.where(node_ids == dstv[...], 1.0, 0.0)
        out_ref[...] += oh @ msgv[...]
        return carry

    lax.fori_loop(c0, c1, chunk, 0)


def _window_call(estart, dstc, msg):
    return pl.pallas_call(
        _window_body,
        grid_spec=pltpu.PrefetchScalarGridSpec(
            num_scalar_prefetch=1,
            grid=(NWIN,),
            in_specs=[
                pl.BlockSpec(memory_space=pltpu.MemorySpace.HBM),
                pl.BlockSpec(memory_space=pltpu.MemorySpace.HBM),
            ],
            out_specs=pl.BlockSpec((WN, FP), lambda w, s: (w, 0)),
            scratch_shapes=[
                pltpu.VMEM((1, CH2), jnp.int32),
                pltpu.VMEM((CH2, FP), jnp.float32),
                pltpu.SemaphoreType.DMA,
                pltpu.SemaphoreType.DMA,
            ],
        ),
        out_shape=jax.ShapeDtypeStruct((NWIN * WN, FP), jnp.float32),
    )(estart, dstc, msg)


def _final_body(in_ref, wct_ref, bias_ref, out_ref):
    out_ref[...] = in_ref[...] @ wct_ref[...] + bias_ref[...]


def _final_call(out_pad, wct, bias_row):
    grid = (NWIN * WN // NB,)
    return pl.pallas_call(
        _final_body,
        grid=grid,
        in_specs=[
            pl.BlockSpec((NB, FP), lambda b: (b, 0)),
            pl.BlockSpec((FP, FP), lambda b: (0, 0)),
            pl.BlockSpec((1, FP), lambda b: (0, 0)),
        ],
        out_specs=pl.BlockSpec((NB, FP), lambda b: (b, 0)),
        out_shape=jax.ShapeDtypeStruct((NWIN * WN, FP), jnp.float32),
    )(out_pad, wct, bias_row)


def kernel(x, edge_index, edge_vec, w1, b1, w2, b2, w3, b3, lw0, lw1, lw2, lb):
    f32 = jnp.float32
    x2 = jnp.pad(x.reshape(N_NODES, F), ((0, 0), (0, FP - F)))
    src = edge_index[0]
    dst = edge_index[1]
    npad = E_PAD - src.shape[0]
    src_p = jnp.concatenate([src, jnp.zeros((npad,), jnp.int32)])
    ev_p = jnp.concatenate(
        [edge_vec,
         jnp.tile(jnp.asarray([[1e3, 0.0, 0.0]], f32), (npad, 1))])
    dst_p = jnp.concatenate([dst, jnp.full((npad,), 2 * N_NODES, jnp.int32)])

    order = jnp.argsort(dst_p)
    src_s = src_p[order]
    ev_s = ev_p[order]
    dst_s = dst_p[order]
    estart = jnp.searchsorted(
        dst_s, jnp.arange(NWIN + 1, dtype=jnp.int32) * WN).astype(jnp.int32)
    dstc = dst_s.reshape(NCHUNK, CH2)
    src2 = src_s.reshape(NW, B_PER_W)

    w3p = w3.reshape(OUT_F, IN_F, 64).transpose(1, 0, 2).reshape(1024, 64)
    b3p = b3.reshape(OUT_F, IN_F).T.reshape(1, 1024)
    tm = jnp.asarray(_TM)
    rm = jnp.asarray(_RM)
    e0 = jnp.zeros((N_HARM,), f32).at[0].set(1.0)
    e1 = jnp.zeros((N_HARM,), f32).at[1:4].set(1.0)
    e2 = jnp.zeros((N_HARM,), f32).at[4:9].set(1.0)
    wc = (jnp.kron(lw0, jnp.diag(e0)) + jnp.kron(lw1, jnp.diag(e1))
          + jnp.kron(lw2, jnp.diag(e2)))
    wcp = jnp.zeros((FP, FP), f32).at[:F, :F].set(wc)
    bias_row = jnp.zeros((FP,), f32).at[jnp.arange(OUT_F) * N_HARM].set(lb)

    x_src = _gather_call(x2, src2)
    msg = _dense_call(ev_s, x_src, w1.T, b1.reshape(1, 64), w2.T,
                      b2.reshape(1, 64), w3p.T, b3p, tm, rm)
    out_pad = _window_call(estart, dstc, msg)
    res_pad = _final_call(out_pad, wcp.T, bias_row.reshape(1, FP))
    return res_pad[:N_NODES, :F].reshape(N_NODES, OUT_F, N_HARM)


# speedup vs baseline: 5.4230x; 1.0075x over previous
import functools

import jax
import jax.numpy as jnp
import numpy as np
from jax import lax
from jax.experimental import pallas as pl
from jax.experimental.pallas import tpu as pltpu
from jax.experimental.pallas import tpu_sc as plsc

N_NODES = 10000
IN_F = 32
OUT_F = 32
N_HARM = 9
F = IN_F * N_HARM
FP = 384
N_RBF = 16
CUTOFF = 5.0

NC = 2
NS = 16
NW = NC * NS

E_PAD = 102400
B_PER_W = E_PAD // NW
GCH = 128

EB = 1024
NB = 1024

WN = 512
NWIN = 10240 // WN
CH2 = 1024
NCHUNK = E_PAD // CH2

_SH_C0 = 0.28209479177387814
_SH_C1 = 0.4886025119029199
_SH_C2 = 1.0925484305920792
_SH_C20 = 0.31539156525252005
_SH_C22 = 0.5462742152960396

_TM = np.zeros((N_HARM, FP), np.float32)
for _i in range(IN_F):
    for _h in range(N_HARM):
        _TM[_h, _i * N_HARM + _h] = 1.0
_RM = np.zeros((OUT_F, FP), np.float32)
for _o in range(OUT_F):
    for _h in range(N_HARM):
        _RM[_o, _o * N_HARM + _h] = 1.0


def _gather_body(x_hbm, src_hbm, out_hbm, idx_v, rows_v, sem):
    wid = lax.axis_index("s") * NC + lax.axis_index("c")
    base = wid * B_PER_W
    pltpu.sync_copy(src_hbm.at[wid], idx_v)

    def body(cc, carry):
        pltpu.async_copy(
            x_hbm.at[idx_v.at[pl.ds(cc * GCH, GCH)]], rows_v, sem
        ).wait()
        pltpu.sync_copy(rows_v, out_hbm.at[pl.ds(base + cc * GCH, GCH)])
        return carry

    lax.fori_loop(0, B_PER_W // GCH, body, 0)


_gather_call = functools.partial(
    pl.kernel,
    out_type=jax.ShapeDtypeStruct((E_PAD, FP), jnp.float32),
    mesh=plsc.VectorSubcoreMesh(
        core_axis_name="c", subcore_axis_name="s", num_cores=NC, num_subcores=NS
    ),
    scratch_types=[
        pltpu.VMEM((B_PER_W,), jnp.int32),
        pltpu.VMEM((GCH, FP), jnp.float32),
        pltpu.SemaphoreType.DMA,
    ],
)(_gather_body)


def _dense_body(ev_ref, xsrc_ref, w1t_ref, b1_ref, w2t_ref, b2_ref, w3pt_ref,
                b3p_ref, tm_ref, rm_ref, tm_f_ref, out_ref):
    ev = ev_ref[...]
    d2 = jnp.sum(ev * ev, axis=1, keepdims=True)
    dd = jnp.sqrt(d2)
    nv = ev / (dd + 1e-12)
    xn, yn, zn = nv[:, 0:1], nv[:, 1:2], nv[:, 2:3]
    sh = jnp.concatenate(
        [
            jnp.full_like(xn, _SH_C0),
            _SH_C1 * yn, _SH_C1 * zn, _SH_C1 * xn,
            _SH_C2 * xn * yn, _SH_C2 * yn * zn,
            _SH_C20 * (3.0 * zn * zn - 1.0),
            _SH_C2 * xn * zn, _SH_C22 * (xn * xn - yn * yn),
        ],
        axis=1,
    )
    centers = lax.broadcasted_iota(jnp.int32, (1, N_RBF), 1).astype(
        jnp.float32) * (CUTOFF / (N_RBF - 1))
    width = CUTOFF / N_RBF
    rbf = jnp.exp(-((dd - centers) ** 2) * (1.0 / (2.0 * width * width)))
    cut = 0.5 * (1.0 + jnp.cos(dd * (np.pi / CUTOFF)))
    cut = jnp.where(dd < CUTOFF, cut, 0.0)

    h1 = rbf @ w1t_ref[...] + b1_ref[...]
    h1 = h1 * (1.0 / (1.0 + jnp.exp(-h1)))
    h2 = h1 @ w2t_ref[...] + b2_ref[...]
    h2 = h2 * (1.0 / (1.0 + jnp.exp(-h2)))
    bf16 = jnp.bfloat16
    rw = jax.lax.dot(h2.astype(bf16), w3pt_ref[...],
                     preferred_element_type=jnp.float32) + b3p_ref[...]
    rw = rw * cut

    tm = tm_ref[...]
    rm = rm_ref[...]
    sh288 = jax.lax.dot(sh.astype(bf16), tm,
                        preferred_element_type=jnp.float32)
    rwb = rw.astype(bf16)
    xsb = xsrc_ref[...].astype(bf16)

    acc = jnp.zeros((EB, FP), jnp.float32)
    for i in range(IN_F):
        a = jax.lax.dot(rwb[:, i * OUT_F:(i + 1) * OUT_F], rm,
                        preferred_element_type=jnp.float32)
        b = jax.lax.dot(xsb[:, i * N_HARM:(i + 1) * N_HARM], tm,
                        preferred_element_type=jnp.float32)
        acc = acc + a * b
    out_ref[...] = acc * sh288


def _dense_call(ev_p, x_src, w1t, b1r, w2t, b2r, w3pt, b3pr, tm, rm, tmf):
    grid = (E_PAD // EB,)
    full = lambda shape: pl.BlockSpec(shape, lambda b: (0, 0))
    return pl.pallas_call(
        _dense_body,
        grid=grid,
        in_specs=[
            pl.BlockSpec((EB, 3), lambda b: (b, 0)),
            pl.BlockSpec((EB, FP), lambda b: (b, 0)),
            full((N_RBF, 64)),
            full((1, 64)),
            full((64, 64)),
            full((1, 64)),
            full((64, 1024)),
            full((1, 1024)),
            full((N_HARM, FP)),
            full((OUT_F, FP)),
            full((N_HARM, FP)),
        ],
        out_specs=pl.BlockSpec((EB, FP), lambda b: (b, 0)),
        out_shape=jax.ShapeDtypeStruct((E_PAD, FP), jnp.float32),
    )(ev_p, x_src, w1t, b1r, w2t, b2r, w3pt, b3pr, tm, rm, tmf)


def _window_body(estart_ref, dstc_hbm, msg_hbm, out_ref, dstv, msgv, sem_d,
                 sem_m):
    w = pl.program_id(0)
    e0 = estart_ref[w]
    e1 = estart_ref[w + 1]
    c0 = e0 // CH2
    c1 = lax.div(e1 + CH2 - 1, CH2)
    base = w * WN
    out_ref[...] = jnp.zeros((WN, FP), jnp.float32)

    def chunk(c, carry):
        cp_d = pltpu.make_async_copy(dstc_hbm.at[pl.ds(c, 1)], dstv, sem_d)
        cp_m = pltpu.make_async_copy(
            msg_hbm.at[pl.ds(c * CH2, CH2)], msgv, sem_m)
        cp_d.start()
        cp_m.start()
        cp_d.wait()
        cp_m.wait()
        node_ids = lax.broadcasted_iota(jnp.int32, (WN, CH2), 0) + base
        oh = jnp.where(node_ids == dstv[...], 1.0, 0.0)
        out_ref[...] += oh @ msgv[...]
        return carry

    lax.fori_loop(c0, c1, chunk, 0)


def _window_call(estart, dstc, msg):
    return pl.pallas_call(
        _window_body,
        grid_spec=pltpu.PrefetchScalarGridSpec(
            num_scalar_prefetch=1,
            grid=(NWIN,),
            in_specs=[
                pl.BlockSpec(memory_space=pltpu.MemorySpace.HBM),
                pl.BlockSpec(memory_space=pltpu.MemorySpace.HBM),
            ],
            out_specs=pl.BlockSpec((WN, FP), lambda w, s: (w, 0)),
            scratch_shapes=[
                pltpu.VMEM((1, CH2), jnp.int32),
                pltpu.VMEM((CH2, FP), jnp.float32),
                pltpu.SemaphoreType.DMA,
                pltpu.SemaphoreType.DMA,
            ],
        ),
        out_shape=jax.ShapeDtypeStruct((NWIN * WN, FP), jnp.float32),
    )(estart, dstc, msg)


def _final_body(in_ref, wct_ref, bias_ref, out_ref):
    out_ref[...] = in_ref[...] @ wct_ref[...] + bias_ref[...]


def _final_call(out_pad, wct, bias_row):
    grid = (NWIN * WN // NB,)
    return pl.pallas_call(
        _final_body,
        grid=grid,
        in_specs=[
            pl.BlockSpec((NB, FP), lambda b: (b, 0)),
            pl.BlockSpec((FP, FP), lambda b: (0, 0)),
            pl.BlockSpec((1, FP), lambda b: (0, 0)),
        ],
        out_specs=pl.BlockSpec((NB, FP), lambda b: (b, 0)),
        out_shape=jax.ShapeDtypeStruct((NWIN * WN, FP), jnp.float32),
    )(out_pad, wct, bias_row)


def kernel(x, edge_index, edge_vec, w1, b1, w2, b2, w3, b3, lw0, lw1, lw2, lb):
    f32 = jnp.float32
    x2 = jnp.pad(x.reshape(N_NODES, F), ((0, 0), (0, FP - F)))
    src = edge_index[0]
    dst = edge_index[1]
    npad = E_PAD - src.shape[0]
    src_p = jnp.concatenate([src, jnp.zeros((npad,), jnp.int32)])
    ev_p = jnp.concatenate(
        [edge_vec,
         jnp.tile(jnp.asarray([[1e3, 0.0, 0.0]], f32), (npad, 1))])
    dst_p = jnp.concatenate([dst, jnp.full((npad,), 2 * N_NODES, jnp.int32)])

    order = jnp.argsort(dst_p)
    src_s = src_p[order]
    ev_s = ev_p[order]
    dst_s = dst_p[order]
    estart = jnp.searchsorted(
        dst_s, jnp.arange(NWIN + 1, dtype=jnp.int32) * WN).astype(jnp.int32)
    dstc = dst_s.reshape(NCHUNK, CH2)
    src2 = src_s.reshape(NW, B_PER_W)

    w3p = w3.reshape(OUT_F, IN_F, 64).transpose(1, 0, 2).reshape(1024, 64)
    b3p = b3.reshape(OUT_F, IN_F).T.reshape(1, 1024)
    tm = jnp.asarray(_TM)
    rm = jnp.asarray(_RM)
    e0 = jnp.zeros((N_HARM,), f32).at[0].set(1.0)
    e1 = jnp.zeros((N_HARM,), f32).at[1:4].set(1.0)
    e2 = jnp.zeros((N_HARM,), f32).at[4:9].set(1.0)
    wc = (jnp.kron(lw0, jnp.diag(e0)) + jnp.kron(lw1, jnp.diag(e1))
          + jnp.kron(lw2, jnp.diag(e2)))
    wcp = jnp.zeros((FP, FP), f32).at[:F, :F].set(wc)
    bias_row = jnp.zeros((FP,), f32).at[jnp.arange(OUT_F) * N_HARM].set(lb)

    x_src = _gather_call(x2, src2)
    msg = _dense_call(ev_s, x_src, w1.T, b1.reshape(1, 64), w2.T,
                      b2.reshape(1, 64), w3p.T.astype(jnp.bfloat16), b3p,
                      tm.astype(jnp.bfloat16), rm.astype(jnp.bfloat16), tm)
    out_pad = _window_call(estart, dstc, msg)
    res_pad = _final_call(out_pad, wcp.T, bias_row.reshape(1, FP))
    return res_pad[:N_NODES, :F].reshape(N_NODES, OUT_F, N_HARM)
